# baseline (speedup 1.0000x reference)
import functools

import jax
import jax.numpy as jnp
from jax import lax
from jax.experimental import pallas as pl
from jax.experimental.pallas import tpu as pltpu
from jax.experimental.pallas import tpu_sc as plsc

BATCH = 16384
FIELDS = 26
EMBED = 32
TOTAL = BATCH * FIELDS
PROWS = 250000
PW = 128
NC = 2
NS = 16
NW = NC * NS
ROWS_W = BATCH // NW
FLAT_W = ROWS_W * FIELDS
CHUNK = 32
NCHUNK = FLAT_W // CHUNK
NOBUF = 2
LANES = 16


def _emb_body(idx_hbm, table_hbm, out_hbm,
              idx_v, flatp_v, flats_v, gath_v, comp_v, *sems):
    gsems = sems[:NOBUF]
    osems = sems[NOBUF:]
    wid = lax.axis_index("s") * NC + lax.axis_index("c")
    base = wid * ROWS_W
    fbase = wid * FLAT_W

    pltpu.sync_copy(idx_hbm.at[pl.ds(base, ROWS_W), :], idx_v)

    lane = lax.iota(jnp.int32, LANES)
    chi = jnp.minimum(lane + LANES, FIELDS - 1)

    def repack_step(r, carry):
        rv = lane * 0 + r
        lo = plsc.load_gather(idx_v, [rv, lane])
        hi = plsc.load_gather(idx_v, [rv, chi])
        pos = rv * FIELDS + lane
        plsc.store_scatter(flatp_v, [pos], lo >> 2)
        plsc.store_scatter(flats_v, [pos], (lo & 3) * EMBED)
        pos2 = pos + LANES
        m2 = lane < (FIELDS - LANES)
        plsc.store_scatter(flatp_v, [pos2], hi >> 2, mask=m2)
        plsc.store_scatter(flats_v, [pos2], (hi & 3) * EMBED, mask=m2)
        return carry

    lax.fori_loop(0, ROWS_W, repack_step, 0)

    def one_chunk(c, b, first):
        pltpu.async_copy(
            table_hbm.at[flatp_v.at[pl.ds(c * CHUNK, CHUNK)]],
            gath_v.at[b], gsems[b])
        pltpu.make_async_copy(
            table_hbm.at[pl.ds(0, CHUNK)], gath_v.at[b], gsems[b]).wait()

        @pl.when(jnp.logical_not(first))
        def _():
            pltpu.make_async_copy(
                out_hbm.at[pl.ds(0, CHUNK)], comp_v.at[b], osems[b]).wait()

        gv = gath_v.at[b]
        cv = comp_v.at[b]

        def extract_group(q, carry):
            sv = flats_v[pl.ds(c * CHUNK + q * LANES, LANES)]
            for j in range(LANES):
                s = sv[j]
                cv[q * LANES + j, pl.ds(0, LANES)] = (
                    gv[q * LANES + j, pl.ds(s, LANES)])
                cv[q * LANES + j, pl.ds(LANES, LANES)] = (
                    gv[q * LANES + j, pl.ds(s + LANES, LANES)])
            return carry

        lax.fori_loop(0, CHUNK // LANES, extract_group, 0)

        pltpu.async_copy(
            comp_v.at[b],
            out_hbm.at[pl.ds(fbase + c * CHUNK, CHUNK)],
            osems[b])

    def super_step(i, carry):
        for b in range(NOBUF):
            one_chunk(i * NOBUF + b, b, i == 0)
        return carry

    lax.fori_loop(0, NCHUNK // NOBUF, super_step, 0)
    for b in range(NOBUF):
        pltpu.make_async_copy(
            out_hbm.at[pl.ds(0, CHUNK)], comp_v.at[b], osems[b]).wait()


@jax.jit
def kernel(x, emb_weight):
    idx = x.astype(jnp.int32)
    table = emb_weight.reshape(PROWS, PW)
    mesh = plsc.VectorSubcoreMesh(core_axis_name="c", subcore_axis_name="s")
    run = functools.partial(
        pl.kernel,
        out_type=jax.ShapeDtypeStruct((TOTAL, EMBED), jnp.float32),
        mesh=mesh,
        scratch_types=[
            pltpu.VMEM((ROWS_W, FIELDS), jnp.int32),
            pltpu.VMEM((FLAT_W,), jnp.int32),
            pltpu.VMEM((FLAT_W,), jnp.int32),
            pltpu.VMEM((NOBUF, CHUNK, PW), jnp.float32),
            pltpu.VMEM((NOBUF, CHUNK, EMBED), jnp.float32),
        ] + [pltpu.SemaphoreType.DMA] * (2 * NOBUF),
        compiler_params=pltpu.CompilerParams(
            use_tc_tiling_on_sc=True, needs_layout_passes=False),
    )(_emb_body)
    out = run(idx, table)
    return out.reshape(BATCH, FIELDS, EMBED)


# speedup vs baseline: 1.6652x; 1.6652x over previous
import functools

import jax
import jax.numpy as jnp
from jax import lax
from jax.experimental import pallas as pl
from jax.experimental import layout
from jax.experimental.pallas import tpu as pltpu
from jax.experimental.pallas import tpu_sc as plsc

BATCH = 16384
FIELDS = 26
EMBED = 32
TOTAL = BATCH * FIELDS
NC = 2
NS = 16
NW = NC * NS
B_PER_W = TOTAL // NW
CHUNK = 1024
NCHUNK = B_PER_W // CHUNK
NBUF = 3


def _emb_body(idx_hbm, table_hbm, out_hbm, idx_v, rows_v, *sems):
    gsems = sems[:NBUF]
    osems = sems[NBUF:]
    wid = lax.axis_index("s") * NC + lax.axis_index("c")
    base = wid * B_PER_W

    pltpu.sync_copy(idx_hbm.at[pl.ds(base, B_PER_W)], idx_v)

    def start_gather(g):
        b = g % NBUF
        return pltpu.async_copy(
            table_hbm.at[idx_v.at[pl.ds(g * CHUNK, CHUNK)]],
            rows_v.at[b], gsems[b])

    gds = [None] * NCHUNK
    ods = [None] * NCHUNK
    for g in range(min(NBUF, NCHUNK)):
        gds[g] = start_gather(g)
    for g in range(NCHUNK):
        b = g % NBUF
        gds[g].wait()
        ods[g] = pltpu.async_copy(
            rows_v.at[b],
            out_hbm.at[pl.ds(base + g * CHUNK, CHUNK)],
            osems[b])
        nxt = g + NBUF
        if nxt < NCHUNK:
            ods[g].wait()
            gds[nxt] = start_gather(nxt)
    for g in range(max(NCHUNK - NBUF, 0), NCHUNK):
        ods[g].wait()


def _kernel_impl(x, emb_weight):
    idx = x.astype(jnp.int32).reshape(TOTAL)
    mesh = plsc.VectorSubcoreMesh(core_axis_name="c", subcore_axis_name="s")
    run = functools.partial(
        pl.kernel,
        out_type=jax.ShapeDtypeStruct((TOTAL, EMBED), jnp.float32),
        mesh=mesh,
        scratch_types=[
            pltpu.VMEM((B_PER_W,), jnp.int32),
            pltpu.VMEM((NBUF, CHUNK, EMBED), jnp.float32),
        ] + [pltpu.SemaphoreType.DMA] * (2 * NBUF),
        compiler_params=pltpu.CompilerParams(use_tc_tiling_on_sc=False),
    )(_emb_body)
    out = run(idx, emb_weight)
    return out.reshape(BATCH, FIELDS, EMBED)


_jitted = None


def kernel(x, emb_weight):
    global _jitted
    if _jitted is None:
        dev = jax.devices()[0]
        fmt = layout.Format(
            layout.Layout(major_to_minor=(0, 1, 2), tiling=()),
            jax.sharding.SingleDeviceSharding(dev))
        _jitted = jax.jit(_kernel_impl, out_shardings=fmt)
    return _jitted(x, emb_weight)


kernel.__name__ = "_kernel_impl"


# speedup vs baseline: 1.6660x; 1.0005x over previous
import functools

import jax
import jax.numpy as jnp
from jax import lax
from jax.experimental import pallas as pl
from jax.experimental import layout
from jax.experimental.pallas import tpu as pltpu
from jax.experimental.pallas import tpu_sc as plsc

BATCH = 16384
FIELDS = 26
EMBED = 32
VOCAB = 1000000
TOTAL = BATCH * FIELDS
NC = 2
NS = 16
NW = NC * NS
LANES = 16

TS = (VOCAB + 127) // 128
NWIN = TS - 2
WIN_BASE = NWIN // NW
WIN_EXTRA = NWIN - WIN_BASE * NW
TAIL = VOCAB - NWIN * 128
NTBUF = 2

B_PER_W = TOTAL // NW
CHUNK = 1024
NCHUNK = B_PER_W // CHUNK
NBUF = 3


def _detile_body(raw_hbm, tail_hbm, t1_hbm, buf_v, tr_v, *sems):
    ssems = sems[:NTBUF]
    osems = sems[NTBUF:]
    wid = lax.axis_index("s") * NC + lax.axis_index("c")
    extra0 = NW - WIN_EXTRA
    start = wid * WIN_BASE + jnp.maximum(wid - extra0, 0)
    count = jnp.where(wid >= extra0, WIN_BASE + 1, WIN_BASE)
    lane = lax.iota(jnp.int32, LANES)
    g_lo, g_hi = lane // 8, (lane + LANES) // 8
    s_lo, s_hi = (lane % 8) * 4, ((lane + LANES) % 8) * 4

    def one_window(win, b, first):
        for g in range(4):
            pltpu.async_copy(
                raw_hbm.at[pl.ds((g * TS + win) * 32, 32), :],
                buf_v.at[b].at[g], ssems[b])
        for g in range(4):
            pltpu.make_async_copy(
                raw_hbm.at[pl.ds(0, 32), :], buf_v.at[b].at[g],
                ssems[b]).wait()

        @pl.when(jnp.logical_not(first))
        def _():
            pltpu.make_async_copy(
                t1_hbm.at[pl.ds(0, 128)], tr_v.at[b], osems[b]).wait()

        bv = buf_v.at[b]
        tv = tr_v.at[b]

        def row_step(i, carry):
            sh = i // 32
            low = i - sh * 32
            lv = lane * 0 + low
            tv[i, pl.ds(0, LANES)] = plsc.load_gather(
                bv, [g_lo, s_lo + sh, lv])
            tv[i, pl.ds(LANES, LANES)] = plsc.load_gather(
                bv, [g_hi, s_hi + sh, lv])
            return carry

        lax.fori_loop(0, 128, row_step, 0)
        pltpu.async_copy(tr_v.at[b], t1_hbm.at[pl.ds(win * 128, 128)],
                         osems[b])

    def super_step(t, carry):
        for b in range(NTBUF):
            one_window(start + t * NTBUF + b, b, t == 0)
        return carry

    lax.fori_loop(0, WIN_BASE // NTBUF, super_step, 0)

    @pl.when(count > WIN_BASE)
    def _():
        one_window(start + WIN_BASE, 0, False)

    @pl.when(wid == 0)
    def _():
        pltpu.sync_copy(tail_hbm, t1_hbm.at[pl.ds(NWIN * 128, TAIL)])
    for b in range(NTBUF):
        pltpu.make_async_copy(
            t1_hbm.at[pl.ds(0, 128)], tr_v.at[b], osems[b]).wait()


def _gather_body(idx_hbm, table_hbm, out_hbm, idx_v, rows_v, *sems):
    gsems = sems[:NBUF]
    osems = sems[NBUF:]
    wid = lax.axis_index("s") * NC + lax.axis_index("c")
    base = wid * B_PER_W

    pltpu.sync_copy(idx_hbm.at[pl.ds(base, B_PER_W)], idx_v)

    def start_gather(g):
        b = g % NBUF
        return pltpu.async_copy(
            table_hbm.at[idx_v.at[pl.ds(g * CHUNK, CHUNK)]],
            rows_v.at[b], gsems[b])

    gds = [None] * NCHUNK
    ods = [None] * NCHUNK
    for g in range(min(NBUF, NCHUNK)):
        gds[g] = start_gather(g)
    for g in range(NCHUNK):
        b = g % NBUF
        gds[g].wait()
        ods[g] = pltpu.async_copy(
            rows_v.at[b],
            out_hbm.at[pl.ds(base + g * CHUNK, CHUNK)],
            osems[b])
        nxt = g + NBUF
        if nxt < NCHUNK:
            ods[g].wait()
            gds[nxt] = start_gather(nxt)
    for g in range(max(NCHUNK - NBUF, 0), NCHUNK):
        ods[g].wait()


def kernel(x, emb_weight):
    mesh = plsc.VectorSubcoreMesh(core_axis_name="c", subcore_axis_name="s")
    params = pltpu.CompilerParams(
        use_tc_tiling_on_sc=False, needs_layout_passes=False)

    t1 = emb_weight

    idx = x.astype(jnp.int32).reshape(TOTAL)
    gather = functools.partial(
        pl.kernel,
        out_type=jax.ShapeDtypeStruct((TOTAL, EMBED), jnp.float32),
        mesh=mesh,
        scratch_types=[
            pltpu.VMEM((B_PER_W,), jnp.int32),
            pltpu.VMEM((NBUF, CHUNK, EMBED), jnp.float32),
        ] + [pltpu.SemaphoreType.DMA] * (2 * NBUF),
        compiler_params=params,
    )(_gather_body)
    out = gather(idx, t1)
    return out.reshape(BATCH, FIELDS, EMBED)
